# baseline (speedup 1.0000x reference)
import functools

import jax
import jax.numpy as jnp
from jax import lax
from jax.experimental import pallas as pl
from jax.experimental.pallas import tpu as pltpu
from jax.experimental.pallas import tpu_sc as plsc

B = 16384
NC = 2
NS = 16
NW = NC * NS
BPW = B // NW
CH = 16
NP = BPW // (2 * CH)

D_MLP = 32
D_MF = 16
BLK = 2048


def _sc_gather(user_idx, item_idx, up, ip, uf, itf):
    mesh = plsc.VectorSubcoreMesh(core_axis_name="c", subcore_axis_name="s")

    @functools.partial(
        pl.kernel,
        mesh=mesh,
        compiler_params=pltpu.CompilerParams(use_tc_tiling_on_sc=True),
        out_type=jax.ShapeDtypeStruct((B, 128), jnp.float32),
        scratch_types=[
            pltpu.VMEM((BPW,), jnp.int32),
            pltpu.VMEM((BPW,), jnp.int32),
            pltpu.VMEM((CH, 128), jnp.float32),
            pltpu.VMEM((CH, 128), jnp.float32),
            pltpu.VMEM((CH, 128), jnp.float32),
            pltpu.VMEM((CH, 128), jnp.float32),
            pltpu.VMEM((CH, 128), jnp.float32),
            pltpu.VMEM((CH, 128), jnp.float32),
            pltpu.VMEM((CH, 128), jnp.float32),
            pltpu.VMEM((CH, 128), jnp.float32),
            pltpu.VMEM((CH, 128), jnp.float32),
            pltpu.VMEM((CH, 128), jnp.float32),
            pltpu.SemaphoreType.DMA,
            pltpu.SemaphoreType.DMA,
            pltpu.SemaphoreType.DMA,
            pltpu.SemaphoreType.DMA,
        ],
    )
    def k(u_h, i_h, up_h, ip_h, uf_h, if_h, out_h,
          uv, iv,
          bu0, bi0, bf0, bg0, slab0,
          bu1, bi1, bf1, bg1, slab1,
          sem_g0, sem_g1, sem_s0, sem_s1):
        wid = lax.axis_index("s") * NC + lax.axis_index("c")
        base = wid * BPW
        pltpu.sync_copy(u_h.at[pl.ds(base, BPW)], uv)
        pltpu.sync_copy(i_h.at[pl.ds(base, BPW)], iv)

        sets = ((bu0, bi0, bf0, bg0, slab0, sem_g0, sem_s0),
                (bu1, bi1, bf1, bg1, slab1, sem_g1, sem_s1))

        def fire(p, c):
            bu, bi, bf, bg, _, sem_g, _ = sets[p]
            uvec = uv[pl.ds(c * CH, CH)]
            ivec = iv[pl.ds(c * CH, CH)]
            pltpu.async_copy(up_h.at[lax.shift_right_logical(uvec, 2)], bu, sem_g)
            pltpu.async_copy(ip_h.at[lax.shift_right_logical(ivec, 2)], bi, sem_g)
            pltpu.async_copy(uf_h.at[lax.shift_right_logical(uvec, 3)], bf, sem_g)
            pltpu.async_copy(if_h.at[lax.shift_right_logical(ivec, 3)], bg, sem_g)

        def drain_extract(p, c, first):
            bu, bi, bf, bg, slab, sem_g, sem_s = sets[p]
            uvec = uv[pl.ds(c * CH, CH)]
            ivec = iv[pl.ds(c * CH, CH)]
            ou = lax.shift_left(lax.bitwise_and(uvec, 3), 5)
            oi = lax.shift_left(lax.bitwise_and(ivec, 3), 5)
            of = lax.shift_left(lax.bitwise_and(uvec, 7), 4)
            og = lax.shift_left(lax.bitwise_and(ivec, 7), 4)
            pltpu.make_async_copy(up_h.at[pl.ds(0, CH)], bu, sem_g).wait()
            pltpu.make_async_copy(ip_h.at[pl.ds(0, CH)], bi, sem_g).wait()
            pltpu.make_async_copy(uf_h.at[pl.ds(0, CH)], bf, sem_g).wait()
            pltpu.make_async_copy(if_h.at[pl.ds(0, CH)], bg, sem_g).wait()

            @pl.when(jnp.logical_not(first))
            def _():
                pltpu.make_async_copy(
                    slab, out_h.at[pl.ds(pl.multiple_of(base, 8), CH)], sem_s
                ).wait()

            for l in range(CH):
                a = ou[l]
                bq = oi[l]
                f = of[l]
                g = og[l]
                slab[l, pl.ds(0, 16)] = bu[l, pl.ds(a, 16)]
                slab[l, pl.ds(16, 16)] = bu[l, pl.ds(a + 16, 16)]
                slab[l, pl.ds(32, 16)] = bi[l, pl.ds(bq, 16)]
                slab[l, pl.ds(48, 16)] = bi[l, pl.ds(bq + 16, 16)]
                slab[l, pl.ds(64, 16)] = bf[l, pl.ds(f, 16)]
                slab[l, pl.ds(80, 16)] = bg[l, pl.ds(g, 16)]
            row0 = pl.multiple_of(base + c * CH, 8)
            pltpu.async_copy(slab, out_h.at[pl.ds(row0, CH)], sem_s)

        def body(c2, _):
            first = c2 == 0
            fire(0, c2 * 2)
            fire(1, c2 * 2 + 1)
            drain_extract(0, c2 * 2, first)
            drain_extract(1, c2 * 2 + 1, first)
            return 0

        lax.fori_loop(0, NP, body, 0)
        pltpu.make_async_copy(
            slab0, out_h.at[pl.ds(pl.multiple_of(base, 8), CH)], sem_s0
        ).wait()
        pltpu.make_async_copy(
            slab1, out_h.at[pl.ds(pl.multiple_of(base, 8), CH)], sem_s1
        ).wait()

    return k(user_idx, item_idx, up, ip, uf, itf)


def _mlp_body(x, w1, b1, w2, b2, w3, b3, wo3, womf, bo, out):
    xb = x[...]
    h = jnp.dot(xb[:, 0:64], w1[...], preferred_element_type=jnp.float32)
    h = jnp.maximum(h + b1[...], 0.0)
    h = jnp.maximum(jnp.dot(h, w2[...], preferred_element_type=jnp.float32) + b2[...], 0.0)
    h = jnp.maximum(jnp.dot(h, w3[...], preferred_element_type=jnp.float32) + b3[...], 0.0)
    z = jnp.dot(h, wo3[...], preferred_element_type=jnp.float32)
    mf = xb[:, 64:80] * xb[:, 80:96]
    z = z + jnp.dot(mf, womf[...], preferred_element_type=jnp.float32)
    out[...] = jax.nn.sigmoid(z + bo[...])


def _tc_mlp(x, W1, b1, W2, b2, W3, b3, Wo, bo):
    wo3 = Wo[:8]
    womf = Wo[8:]
    b1r = b1.reshape(1, -1)
    b2r = b2.reshape(1, -1)
    b3r = b3.reshape(1, -1)
    bor = bo.reshape(1, -1)

    def full(a):
        return pl.BlockSpec(a.shape, lambda i: (0, 0))

    return pl.pallas_call(
        _mlp_body,
        grid=(B // BLK,),
        in_specs=[
            pl.BlockSpec((BLK, 128), lambda i: (i, 0)),
            full(W1), full(b1r), full(W2), full(b2r),
            full(W3), full(b3r), full(wo3), full(womf), full(bor),
        ],
        out_specs=pl.BlockSpec((BLK, 1), lambda i: (i, 0)),
        out_shape=jax.ShapeDtypeStruct((B, 1), jnp.float32),
    )(x, W1, b1r, W2, b2r, W3, b3r, wo3, womf, bor)


def kernel(user_indices, item_indices, U_mlp, I_mlp, U_mf, I_mf,
           W1, b1, W2, b2, W3, b3, Wo, bo):
    up = U_mlp.reshape(-1, 128)
    ip = I_mlp.reshape(-1, 128)
    uf = U_mf.reshape(-1, 128)
    itf = I_mf.reshape(-1, 128)
    x = _sc_gather(user_indices, item_indices, up, ip, uf, itf)
    return _tc_mlp(x, W1, b1, W2, b2, W3, b3, Wo, bo)


# speedup vs baseline: 1.2758x; 1.2758x over previous
import functools

import jax
import jax.numpy as jnp
from jax import lax
from jax.experimental import pallas as pl
from jax.experimental.pallas import tpu as pltpu
from jax.experimental.pallas import tpu_sc as plsc

B = 16384
NC = 2
NS = 16
NW = NC * NS
BPW = B // NW
CH = 16
NP = BPW // (2 * CH)

D_MLP = 32
D_MF = 16
BLK = 2048


def _sc_gather(user_idx, item_idx, up, ip, uf, itf):
    mesh = plsc.VectorSubcoreMesh(core_axis_name="c", subcore_axis_name="s")

    @functools.partial(
        pl.kernel,
        mesh=mesh,
        compiler_params=pltpu.CompilerParams(use_tc_tiling_on_sc=True),
        out_type=jax.ShapeDtypeStruct((B, 128), jnp.float32),
        scratch_types=[
            pltpu.VMEM((BPW,), jnp.int32),
            pltpu.VMEM((BPW,), jnp.int32),
            pltpu.VMEM((CH, 128), jnp.float32),
            pltpu.VMEM((CH, 128), jnp.float32),
            pltpu.VMEM((CH, 128), jnp.float32),
            pltpu.VMEM((CH, 128), jnp.float32),
            pltpu.VMEM((CH, 128), jnp.float32),
            pltpu.VMEM((CH, 128), jnp.float32),
            pltpu.VMEM((CH, 128), jnp.float32),
            pltpu.VMEM((CH, 128), jnp.float32),
            pltpu.VMEM((CH, 128), jnp.float32),
            pltpu.VMEM((CH, 128), jnp.float32),
            pltpu.SemaphoreType.DMA,
            pltpu.SemaphoreType.DMA,
            pltpu.SemaphoreType.DMA,
            pltpu.SemaphoreType.DMA,
        ],
    )
    def k(u_h, i_h, up_h, ip_h, uf_h, if_h, out_h,
          uv, iv,
          bu0, bi0, bf0, bg0, slab0,
          bu1, bi1, bf1, bg1, slab1,
          sem_g0, sem_g1, sem_s0, sem_s1):
        wid = lax.axis_index("s") * NC + lax.axis_index("c")
        base = wid * BPW
        pltpu.sync_copy(u_h.at[pl.ds(base, BPW)], uv)
        pltpu.sync_copy(i_h.at[pl.ds(base, BPW)], iv)

        sets = ((bu0, bi0, bf0, bg0, slab0, sem_g0, sem_s0),
                (bu1, bi1, bf1, bg1, slab1, sem_g1, sem_s1))

        def fire(p, c):
            bu, bi, bf, bg, _, sem_g, _ = sets[p]
            uvec = uv[pl.ds(c * CH, CH)]
            ivec = iv[pl.ds(c * CH, CH)]

            def prow(v, qw_log):
                hi = lax.shift_left(lax.shift_right_logical(v, 11), qw_log)
                return lax.bitwise_or(hi, lax.bitwise_and(v, (1 << qw_log) - 1))

            pltpu.async_copy(up_h.at[prow(uvec, 9)], bu, sem_g)
            pltpu.async_copy(ip_h.at[prow(ivec, 9)], bi, sem_g)
            pltpu.async_copy(uf_h.at[prow(uvec, 8)], bf, sem_g)
            pltpu.async_copy(if_h.at[prow(ivec, 8)], bg, sem_g)

        def drain_extract(p, c, first):
            bu, bi, bf, bg, slab, sem_g, sem_s = sets[p]
            uvec = uv[pl.ds(c * CH, CH)]
            ivec = iv[pl.ds(c * CH, CH)]
            ou = lax.shift_left(
                lax.bitwise_and(lax.shift_right_logical(uvec, 9), 3), 5)
            oi = lax.shift_left(
                lax.bitwise_and(lax.shift_right_logical(ivec, 9), 3), 5)
            of = lax.shift_left(
                lax.bitwise_and(lax.shift_right_logical(uvec, 8), 7), 4)
            og = lax.shift_left(
                lax.bitwise_and(lax.shift_right_logical(ivec, 8), 7), 4)
            pltpu.make_async_copy(up_h.at[pl.ds(0, CH)], bu, sem_g).wait()
            pltpu.make_async_copy(ip_h.at[pl.ds(0, CH)], bi, sem_g).wait()
            pltpu.make_async_copy(uf_h.at[pl.ds(0, CH)], bf, sem_g).wait()
            pltpu.make_async_copy(if_h.at[pl.ds(0, CH)], bg, sem_g).wait()

            @pl.when(jnp.logical_not(first))
            def _():
                pltpu.make_async_copy(
                    slab, out_h.at[pl.ds(pl.multiple_of(base, 8), CH)], sem_s
                ).wait()

            for l in range(CH):
                a = ou[l]
                bq = oi[l]
                f = of[l]
                g = og[l]
                slab[l, pl.ds(0, 16)] = bu[l, pl.ds(a, 16)]
                slab[l, pl.ds(16, 16)] = bu[l, pl.ds(a + 16, 16)]
                slab[l, pl.ds(32, 16)] = bi[l, pl.ds(bq, 16)]
                slab[l, pl.ds(48, 16)] = bi[l, pl.ds(bq + 16, 16)]
                slab[l, pl.ds(64, 16)] = bf[l, pl.ds(f, 16)]
                slab[l, pl.ds(80, 16)] = bg[l, pl.ds(g, 16)]
            row0 = pl.multiple_of(base + c * CH, 8)
            pltpu.async_copy(slab, out_h.at[pl.ds(row0, CH)], sem_s)

        def body(c2, _):
            first = c2 == 0
            fire(0, c2 * 2)
            fire(1, c2 * 2 + 1)
            drain_extract(0, c2 * 2, first)
            drain_extract(1, c2 * 2 + 1, first)
            return 0

        lax.fori_loop(0, NP, body, 0)
        pltpu.make_async_copy(
            slab0, out_h.at[pl.ds(pl.multiple_of(base, 8), CH)], sem_s0
        ).wait()
        pltpu.make_async_copy(
            slab1, out_h.at[pl.ds(pl.multiple_of(base, 8), CH)], sem_s1
        ).wait()

    return k(user_idx, item_idx, up, ip, uf, itf)


def _mlp_body(x, w1, b1, w2, b2, w3, b3, wo3, womf, bo, out):
    xb = x[...]
    h = jnp.dot(xb[:, 0:64], w1[...], preferred_element_type=jnp.float32)
    h = jnp.maximum(h + b1[...], 0.0)
    h = jnp.maximum(jnp.dot(h, w2[...], preferred_element_type=jnp.float32) + b2[...], 0.0)
    h = jnp.maximum(jnp.dot(h, w3[...], preferred_element_type=jnp.float32) + b3[...], 0.0)
    z = jnp.dot(h, wo3[...], preferred_element_type=jnp.float32)
    mf = xb[:, 64:80] * xb[:, 80:96]
    z = z + jnp.dot(mf, womf[...], preferred_element_type=jnp.float32)
    out[...] = jax.nn.sigmoid(z + bo[...])


def _tc_mlp(x, W1, b1, W2, b2, W3, b3, Wo, bo):
    wo3 = Wo[:8]
    womf = Wo[8:]
    b1r = b1.reshape(1, -1)
    b2r = b2.reshape(1, -1)
    b3r = b3.reshape(1, -1)
    bor = bo.reshape(1, -1)

    def full(a):
        return pl.BlockSpec(a.shape, lambda i: (0, 0))

    return pl.pallas_call(
        _mlp_body,
        grid=(B // BLK,),
        in_specs=[
            pl.BlockSpec((BLK, 128), lambda i: (i, 0)),
            full(W1), full(b1r), full(W2), full(b2r),
            full(W3), full(b3r), full(wo3), full(womf), full(bor),
        ],
        out_specs=pl.BlockSpec((BLK, 1), lambda i: (i, 0)),
        out_shape=jax.ShapeDtypeStruct((B, 1), jnp.float32),
    )(x, W1, b1r, W2, b2r, W3, b3r, wo3, womf, bor)


PW = 2048


def _pack_body(x, out):
    xb = x[...]
    d = xb.shape[0]
    q = 128 // d
    qw = PW // q
    z = jnp.concatenate(
        [xb[:, i * qw:(i + 1) * qw] for i in range(q)], axis=0)
    out[...] = jnp.swapaxes(z, 0, 1)


def _pack(table):
    n, d = table.shape
    tt = table.T
    nw = -(-n // PW)
    return pl.pallas_call(
        _pack_body,
        grid=(nw,),
        in_specs=[pl.BlockSpec((d, PW), lambda i: (0, i))],
        out_specs=pl.BlockSpec((PW * d // 128, 128), lambda i: (i, 0)),
        out_shape=jax.ShapeDtypeStruct((nw * PW * d // 128, 128), jnp.float32),
    )(tt)


def kernel(user_indices, item_indices, U_mlp, I_mlp, U_mf, I_mf,
           W1, b1, W2, b2, W3, b3, Wo, bo):
    up = _pack(U_mlp)
    ip = _pack(I_mlp)
    uf = _pack(U_mf)
    itf = _pack(I_mf)
    x = _sc_gather(user_indices, item_indices, up, ip, uf, itf)
    return _tc_mlp(x, W1, b1, W2, b2, W3, b3, Wo, bo)


# speedup vs baseline: 3.7507x; 2.9398x over previous
import functools

import jax
import jax.numpy as jnp
from jax import lax
from jax.experimental import pallas as pl
from jax.experimental.pallas import tpu as pltpu
from jax.experimental.pallas import tpu_sc as plsc

B = 16384
NC = 2
NS = 16
NW = NC * NS
BPW = B // NW
CH = 16
NP = BPW // (2 * CH)

D_MLP = 32
D_MF = 16
BLK = 2048


def _sc_gather(user_idx, item_idx, up, ip, uf, itf):
    mesh = plsc.VectorSubcoreMesh(core_axis_name="c", subcore_axis_name="s")

    @functools.partial(
        pl.kernel,
        mesh=mesh,
        compiler_params=pltpu.CompilerParams(use_tc_tiling_on_sc=True),
        out_type=jax.ShapeDtypeStruct((B, 128), jnp.float32),
        scratch_types=[
            pltpu.VMEM((BPW,), jnp.int32),
            pltpu.VMEM((BPW,), jnp.int32),
            pltpu.VMEM((CH, 128), jnp.float32),
            pltpu.VMEM((CH, 128), jnp.float32),
            pltpu.VMEM((CH, 128), jnp.float32),
            pltpu.VMEM((CH, 128), jnp.float32),
            pltpu.VMEM((CH, 128), jnp.float32),
            pltpu.VMEM((CH, 128), jnp.float32),
            pltpu.VMEM((CH, 128), jnp.float32),
            pltpu.VMEM((CH, 128), jnp.float32),
            pltpu.VMEM((CH, 128), jnp.float32),
            pltpu.VMEM((CH, 128), jnp.float32),
            pltpu.SemaphoreType.DMA,
            pltpu.SemaphoreType.DMA,
            pltpu.SemaphoreType.DMA,
            pltpu.SemaphoreType.DMA,
        ],
    )
    def k(u_h, i_h, up_h, ip_h, uf_h, if_h, out_h,
          uv, iv,
          bu0, bi0, bf0, bg0, slab0,
          bu1, bi1, bf1, bg1, slab1,
          sem_g0, sem_g1, sem_s0, sem_s1):
        wid = lax.axis_index("s") * NC + lax.axis_index("c")
        base = wid * BPW
        pltpu.sync_copy(u_h.at[pl.ds(base, BPW)], uv)
        pltpu.sync_copy(i_h.at[pl.ds(base, BPW)], iv)

        sets = ((bu0, bi0, bf0, bg0, slab0, sem_g0, sem_s0),
                (bu1, bi1, bf1, bg1, slab1, sem_g1, sem_s1))

        def fire(p, c):
            bu, bi, bf, bg, _, sem_g, _ = sets[p]
            uvec = uv[pl.ds(c * CH, CH)]
            ivec = iv[pl.ds(c * CH, CH)]

            def prow(v, qw_log):
                hi = lax.shift_left(lax.shift_right_logical(v, 13), qw_log)
                return lax.bitwise_or(hi, lax.bitwise_and(v, (1 << qw_log) - 1))

            pltpu.async_copy(up_h.at[prow(uvec, 11)], bu, sem_g)
            pltpu.async_copy(ip_h.at[prow(ivec, 11)], bi, sem_g)
            pltpu.async_copy(uf_h.at[prow(uvec, 10)], bf, sem_g)
            pltpu.async_copy(if_h.at[prow(ivec, 10)], bg, sem_g)

        def drain_extract(p, c, first):
            bu, bi, bf, bg, slab, sem_g, sem_s = sets[p]
            uvec = uv[pl.ds(c * CH, CH)]
            ivec = iv[pl.ds(c * CH, CH)]
            ou = lax.shift_left(
                lax.bitwise_and(lax.shift_right_logical(uvec, 11), 3), 5)
            oi = lax.shift_left(
                lax.bitwise_and(lax.shift_right_logical(ivec, 11), 3), 5)
            of = lax.shift_left(
                lax.bitwise_and(lax.shift_right_logical(uvec, 10), 7), 4)
            og = lax.shift_left(
                lax.bitwise_and(lax.shift_right_logical(ivec, 10), 7), 4)
            pltpu.make_async_copy(up_h.at[pl.ds(0, CH)], bu, sem_g).wait()
            pltpu.make_async_copy(ip_h.at[pl.ds(0, CH)], bi, sem_g).wait()
            pltpu.make_async_copy(uf_h.at[pl.ds(0, CH)], bf, sem_g).wait()
            pltpu.make_async_copy(if_h.at[pl.ds(0, CH)], bg, sem_g).wait()

            @pl.when(jnp.logical_not(first))
            def _():
                pltpu.make_async_copy(
                    slab, out_h.at[pl.ds(pl.multiple_of(base, 8), CH)], sem_s
                ).wait()

            for l in range(CH):
                a = ou[l]
                bq = oi[l]
                f = of[l]
                g = og[l]
                slab[l, pl.ds(0, 16)] = bu[l, pl.ds(a, 16)]
                slab[l, pl.ds(16, 16)] = bu[l, pl.ds(a + 16, 16)]
                slab[l, pl.ds(32, 16)] = bi[l, pl.ds(bq, 16)]
                slab[l, pl.ds(48, 16)] = bi[l, pl.ds(bq + 16, 16)]
                slab[l, pl.ds(64, 16)] = bf[l, pl.ds(f, 16)]
                slab[l, pl.ds(80, 16)] = bg[l, pl.ds(g, 16)]
            row0 = pl.multiple_of(base + c * CH, 8)
            pltpu.async_copy(slab, out_h.at[pl.ds(row0, CH)], sem_s)

        def body(c2, _):
            first = c2 == 0
            fire(0, c2 * 2)
            fire(1, c2 * 2 + 1)
            drain_extract(0, c2 * 2, first)
            drain_extract(1, c2 * 2 + 1, first)
            return 0

        lax.fori_loop(0, NP, body, 0)
        pltpu.make_async_copy(
            slab0, out_h.at[pl.ds(pl.multiple_of(base, 8), CH)], sem_s0
        ).wait()
        pltpu.make_async_copy(
            slab1, out_h.at[pl.ds(pl.multiple_of(base, 8), CH)], sem_s1
        ).wait()

    return k(user_idx, item_idx, up, ip, uf, itf)


def _mlp_body(x, w1, b1, w2, b2, w3, b3, wo3, womf, bo, out):
    xb = x[...]
    h = jnp.dot(xb[:, 0:64], w1[...], preferred_element_type=jnp.float32)
    h = jnp.maximum(h + b1[...], 0.0)
    h = jnp.maximum(jnp.dot(h, w2[...], preferred_element_type=jnp.float32) + b2[...], 0.0)
    h = jnp.maximum(jnp.dot(h, w3[...], preferred_element_type=jnp.float32) + b3[...], 0.0)
    z = jnp.dot(h, wo3[...], preferred_element_type=jnp.float32)
    mf = xb[:, 64:80] * xb[:, 80:96]
    z = z + jnp.dot(mf, womf[...], preferred_element_type=jnp.float32)
    out[...] = jax.nn.sigmoid(z + bo[...])


def _tc_mlp(x, W1, b1, W2, b2, W3, b3, Wo, bo):
    wo3 = Wo[:8]
    womf = Wo[8:]
    b1r = b1.reshape(1, -1)
    b2r = b2.reshape(1, -1)
    b3r = b3.reshape(1, -1)
    bor = bo.reshape(1, -1)

    def full(a):
        return pl.BlockSpec(a.shape, lambda i: (0, 0))

    return pl.pallas_call(
        _mlp_body,
        grid=(B // BLK,),
        in_specs=[
            pl.BlockSpec((BLK, 128), lambda i: (i, 0)),
            full(W1), full(b1r), full(W2), full(b2r),
            full(W3), full(b3r), full(wo3), full(womf), full(bor),
        ],
        out_specs=pl.BlockSpec((BLK, 1), lambda i: (i, 0)),
        out_shape=jax.ShapeDtypeStruct((B, 1), jnp.float32),
    )(x, W1, b1r, W2, b2r, W3, b3r, wo3, womf, bor)


PW = 8192


def _pack_one(xb, out_ref):
    d = xb.shape[0]
    q = 128 // d
    qw = PW // q
    z = jnp.concatenate(
        [xb[:, i * qw:(i + 1) * qw] for i in range(q)], axis=0)
    out_ref[...] = jnp.swapaxes(z, 0, 1)


def _pack_body(xa, xb, outa, outb):
    _pack_one(xa[...], outa)
    _pack_one(xb[...], outb)


def _pack_pair(ta, tb):
    n = ta.shape[0]
    nw = -(-n // PW)
    return pl.pallas_call(
        _pack_body,
        grid=(nw,),
        in_specs=[
            pl.BlockSpec((32, PW), lambda i: (0, i)),
            pl.BlockSpec((16, PW), lambda i: (0, i)),
        ],
        out_specs=[
            pl.BlockSpec((PW * 32 // 128, 128), lambda i: (i, 0)),
            pl.BlockSpec((PW * 16 // 128, 128), lambda i: (i, 0)),
        ],
        out_shape=[
            jax.ShapeDtypeStruct((nw * PW * 32 // 128, 128), jnp.float32),
            jax.ShapeDtypeStruct((nw * PW * 16 // 128, 128), jnp.float32),
        ],
    )(ta.T, tb.T)


def kernel(user_indices, item_indices, U_mlp, I_mlp, U_mf, I_mf,
           W1, b1, W2, b2, W3, b3, Wo, bo):
    up, uf = _pack_pair(U_mlp, U_mf)
    ip, itf = _pack_pair(I_mlp, I_mf)
    x = _sc_gather(user_indices, item_indices, up, ip, uf, itf)
    return _tc_mlp(x, W1, b1, W2, b2, W3, b3, Wo, bo)


# speedup vs baseline: 4.3172x; 1.1510x over previous
import functools

import jax
import jax.numpy as jnp
from jax import lax
from jax.experimental import pallas as pl
from jax.experimental.pallas import tpu as pltpu
from jax.experimental.pallas import tpu_sc as plsc

B = 16384
NC = 2
NS = 16
NW = NC * NS
BPW = B // NW
CH = 16
NP = BPW // (2 * CH)

D_MLP = 32
D_MF = 16
BLK = 2048


def _sc_gather(user_idx, item_idx, up, ip, uf, itf):
    mesh = plsc.VectorSubcoreMesh(core_axis_name="c", subcore_axis_name="s")

    @functools.partial(
        pl.kernel,
        mesh=mesh,
        compiler_params=pltpu.CompilerParams(use_tc_tiling_on_sc=True),
        out_type=jax.ShapeDtypeStruct((B, 128), jnp.float32),
        scratch_types=[
            pltpu.VMEM((BPW,), jnp.int32),
            pltpu.VMEM((BPW,), jnp.int32),
            pltpu.VMEM((CH, 128), jnp.float32),
            pltpu.VMEM((CH, 128), jnp.float32),
            pltpu.VMEM((CH, 128), jnp.float32),
            pltpu.VMEM((CH, 128), jnp.float32),
            pltpu.VMEM((CH, 128), jnp.float32),
            pltpu.VMEM((CH, 128), jnp.float32),
            pltpu.VMEM((CH, 128), jnp.float32),
            pltpu.VMEM((CH, 128), jnp.float32),
            pltpu.VMEM((CH, 128), jnp.float32),
            pltpu.VMEM((CH, 128), jnp.float32),
            pltpu.SemaphoreType.DMA,
            pltpu.SemaphoreType.DMA,
            pltpu.SemaphoreType.DMA,
            pltpu.SemaphoreType.DMA,
        ],
    )
    def k(u_h, i_h, up_h, ip_h, uf_h, if_h, out_h,
          uv, iv,
          bu0, bi0, bf0, bg0, slab0,
          bu1, bi1, bf1, bg1, slab1,
          sem_g0, sem_g1, sem_s0, sem_s1):
        wid = lax.axis_index("s") * NC + lax.axis_index("c")
        base = wid * BPW
        pltpu.sync_copy(u_h.at[pl.ds(base, BPW)], uv)
        pltpu.sync_copy(i_h.at[pl.ds(base, BPW)], iv)

        sets = ((bu0, bi0, bf0, bg0, slab0, sem_g0, sem_s0),
                (bu1, bi1, bf1, bg1, slab1, sem_g1, sem_s1))

        def fire(p, c):
            bu, bi, bf, bg, _, sem_g, _ = sets[p]
            uvec = uv[pl.ds(c * CH, CH)]
            ivec = iv[pl.ds(c * CH, CH)]

            def prow(v, qw_log):
                hi = lax.shift_left(lax.shift_right_logical(v, 14), qw_log)
                return lax.bitwise_or(hi, lax.bitwise_and(v, (1 << qw_log) - 1))

            pltpu.async_copy(up_h.at[prow(uvec, 12)], bu, sem_g)
            pltpu.async_copy(ip_h.at[prow(ivec, 12)], bi, sem_g)
            pltpu.async_copy(uf_h.at[prow(uvec, 11)], bf, sem_g)
            pltpu.async_copy(if_h.at[prow(ivec, 11)], bg, sem_g)

        def drain_extract(p, c, first):
            bu, bi, bf, bg, slab, sem_g, sem_s = sets[p]
            uvec = uv[pl.ds(c * CH, CH)]
            ivec = iv[pl.ds(c * CH, CH)]
            ou = lax.shift_left(
                lax.bitwise_and(lax.shift_right_logical(uvec, 12), 3), 5)
            oi = lax.shift_left(
                lax.bitwise_and(lax.shift_right_logical(ivec, 12), 3), 5)
            of = lax.shift_left(
                lax.bitwise_and(lax.shift_right_logical(uvec, 11), 7), 4)
            og = lax.shift_left(
                lax.bitwise_and(lax.shift_right_logical(ivec, 11), 7), 4)
            pltpu.make_async_copy(up_h.at[pl.ds(0, CH)], bu, sem_g).wait()
            pltpu.make_async_copy(ip_h.at[pl.ds(0, CH)], bi, sem_g).wait()
            pltpu.make_async_copy(uf_h.at[pl.ds(0, CH)], bf, sem_g).wait()
            pltpu.make_async_copy(if_h.at[pl.ds(0, CH)], bg, sem_g).wait()

            @pl.when(jnp.logical_not(first))
            def _():
                pltpu.make_async_copy(
                    slab, out_h.at[pl.ds(pl.multiple_of(base, 8), CH)], sem_s
                ).wait()

            for l in range(CH):
                a = ou[l]
                bq = oi[l]
                f = of[l]
                g = og[l]
                slab[l, pl.ds(0, 16)] = bu[l, pl.ds(a, 16)]
                slab[l, pl.ds(16, 16)] = bu[l, pl.ds(a + 16, 16)]
                slab[l, pl.ds(32, 16)] = bi[l, pl.ds(bq, 16)]
                slab[l, pl.ds(48, 16)] = bi[l, pl.ds(bq + 16, 16)]
                slab[l, pl.ds(64, 16)] = bf[l, pl.ds(f, 16)]
                slab[l, pl.ds(80, 16)] = bg[l, pl.ds(g, 16)]
            row0 = pl.multiple_of(base + c * CH, 8)
            pltpu.async_copy(slab, out_h.at[pl.ds(row0, CH)], sem_s)

        def body(c2, _):
            first = c2 == 0
            fire(0, c2 * 2)
            fire(1, c2 * 2 + 1)
            drain_extract(0, c2 * 2, first)
            drain_extract(1, c2 * 2 + 1, first)
            return 0

        lax.fori_loop(0, NP, body, 0)
        pltpu.make_async_copy(
            slab0, out_h.at[pl.ds(pl.multiple_of(base, 8), CH)], sem_s0
        ).wait()
        pltpu.make_async_copy(
            slab1, out_h.at[pl.ds(pl.multiple_of(base, 8), CH)], sem_s1
        ).wait()

    return k(user_idx, item_idx, up, ip, uf, itf)


def _mlp_body(x, w1, b1, w2, b2, w3, b3, wo3, womf, bo, out):
    xb = x[...]
    h = jnp.dot(xb[:, 0:64], w1[...], preferred_element_type=jnp.float32)
    h = jnp.maximum(h + b1[...], 0.0)
    h = jnp.maximum(jnp.dot(h, w2[...], preferred_element_type=jnp.float32) + b2[...], 0.0)
    h = jnp.maximum(jnp.dot(h, w3[...], preferred_element_type=jnp.float32) + b3[...], 0.0)
    z = jnp.dot(h, wo3[...], preferred_element_type=jnp.float32)
    mf = xb[:, 64:80] * xb[:, 80:96]
    z = z + jnp.dot(mf, womf[...], preferred_element_type=jnp.float32)
    out[...] = jax.nn.sigmoid(z + bo[...])


def _tc_mlp(x, W1, b1, W2, b2, W3, b3, Wo, bo):
    wo3 = Wo[:8]
    womf = Wo[8:]
    b1r = b1.reshape(1, -1)
    b2r = b2.reshape(1, -1)
    b3r = b3.reshape(1, -1)
    bor = bo.reshape(1, -1)

    def full(a):
        return pl.BlockSpec(a.shape, lambda i: (0, 0))

    return pl.pallas_call(
        _mlp_body,
        grid=(B // BLK,),
        in_specs=[
            pl.BlockSpec((BLK, 128), lambda i: (i, 0)),
            full(W1), full(b1r), full(W2), full(b2r),
            full(W3), full(b3r), full(wo3), full(womf), full(bor),
        ],
        out_specs=pl.BlockSpec((BLK, 1), lambda i: (i, 0)),
        out_shape=jax.ShapeDtypeStruct((B, 1), jnp.float32),
    )(x, W1, b1r, W2, b2r, W3, b3r, wo3, womf, bor)


PW = 16384


def _pack_one(xb, out_ref):
    d = xb.shape[0]
    q = 128 // d
    qw = PW // q
    z = jnp.concatenate(
        [xb[:, i * qw:(i + 1) * qw] for i in range(q)], axis=0)
    out_ref[...] = jnp.swapaxes(z, 0, 1)


def _pack_body(xa, xb, outa, outb):
    _pack_one(xa[...], outa)
    _pack_one(xb[...], outb)


def _pack_pair(ta, tb):
    n = ta.shape[0]
    nw = -(-n // PW)
    return pl.pallas_call(
        _pack_body,
        grid=(nw,),
        in_specs=[
            pl.BlockSpec((32, PW), lambda i: (0, i)),
            pl.BlockSpec((16, PW), lambda i: (0, i)),
        ],
        out_specs=[
            pl.BlockSpec((PW * 32 // 128, 128), lambda i: (i, 0)),
            pl.BlockSpec((PW * 16 // 128, 128), lambda i: (i, 0)),
        ],
        out_shape=[
            jax.ShapeDtypeStruct((nw * PW * 32 // 128, 128), jnp.float32),
            jax.ShapeDtypeStruct((nw * PW * 16 // 128, 128), jnp.float32),
        ],
    )(ta.T, tb.T)


def kernel(user_indices, item_indices, U_mlp, I_mlp, U_mf, I_mf,
           W1, b1, W2, b2, W3, b3, Wo, bo):
    up, uf = _pack_pair(U_mlp, U_mf)
    ip, itf = _pack_pair(I_mlp, I_mf)
    x = _sc_gather(user_indices, item_indices, up, ip, uf, itf)
    return _tc_mlp(x, W1, b1, W2, b2, W3, b3, Wo, bo)


# speedup vs baseline: 4.5306x; 1.0494x over previous
import functools

import jax
import jax.numpy as jnp
from jax import lax
from jax.experimental import pallas as pl
from jax.experimental.pallas import tpu as pltpu
from jax.experimental.pallas import tpu_sc as plsc

B = 16384
NC = 2
NS = 16
NW = NC * NS
BPW = B // NW
CH = 16
NP = BPW // (2 * CH)

D_MLP = 32
D_MF = 16
BLK = 2048


def _sc_gather(user_idx, item_idx, up, ip, uf, itf):
    mesh = plsc.VectorSubcoreMesh(core_axis_name="c", subcore_axis_name="s")

    @functools.partial(
        pl.kernel,
        mesh=mesh,
        compiler_params=pltpu.CompilerParams(use_tc_tiling_on_sc=True),
        out_type=jax.ShapeDtypeStruct((B, 128), jnp.float32),
        scratch_types=[
            pltpu.VMEM((BPW,), jnp.int32),
            pltpu.VMEM((BPW,), jnp.int32),
            pltpu.VMEM((CH, 128), jnp.float32),
            pltpu.VMEM((CH, 128), jnp.float32),
            pltpu.VMEM((CH, 128), jnp.float32),
            pltpu.VMEM((CH, 128), jnp.float32),
            pltpu.VMEM((CH, 128), jnp.float32),
            pltpu.VMEM((CH, 128), jnp.float32),
            pltpu.VMEM((CH, 128), jnp.float32),
            pltpu.VMEM((CH, 128), jnp.float32),
            pltpu.VMEM((CH, 128), jnp.float32),
            pltpu.VMEM((CH, 128), jnp.float32),
            pltpu.SemaphoreType.DMA,
            pltpu.SemaphoreType.DMA,
            pltpu.SemaphoreType.DMA,
            pltpu.SemaphoreType.DMA,
        ],
    )
    def k(u_h, i_h, up_h, ip_h, uf_h, if_h, out_h,
          uv, iv,
          bu0, bi0, bf0, bg0, slab0,
          bu1, bi1, bf1, bg1, slab1,
          sem_g0, sem_g1, sem_s0, sem_s1):
        wid = lax.axis_index("s") * NC + lax.axis_index("c")
        base = wid * BPW
        pltpu.sync_copy(u_h.at[pl.ds(base, BPW)], uv)
        pltpu.sync_copy(i_h.at[pl.ds(base, BPW)], iv)

        sets = ((bu0, bi0, bf0, bg0, slab0, sem_g0, sem_s0),
                (bu1, bi1, bf1, bg1, slab1, sem_g1, sem_s1))

        def fire(p, c):
            bu, bi, bf, bg, _, sem_g, _ = sets[p]
            uvec = uv[pl.ds(c * CH, CH)]
            ivec = iv[pl.ds(c * CH, CH)]

            def prow(v, qw_log):
                hi = lax.shift_left(lax.shift_right_logical(v, 15), qw_log)
                return lax.bitwise_or(hi, lax.bitwise_and(v, (1 << qw_log) - 1))

            pltpu.async_copy(up_h.at[prow(uvec, 13)], bu, sem_g)
            pltpu.async_copy(ip_h.at[prow(ivec, 13)], bi, sem_g)
            pltpu.async_copy(uf_h.at[prow(uvec, 12)], bf, sem_g)
            pltpu.async_copy(if_h.at[prow(ivec, 12)], bg, sem_g)

        def drain_extract(p, c, first):
            bu, bi, bf, bg, slab, sem_g, sem_s = sets[p]
            uvec = uv[pl.ds(c * CH, CH)]
            ivec = iv[pl.ds(c * CH, CH)]
            ou = lax.shift_left(
                lax.bitwise_and(lax.shift_right_logical(uvec, 13), 3), 5)
            oi = lax.shift_left(
                lax.bitwise_and(lax.shift_right_logical(ivec, 13), 3), 5)
            of = lax.shift_left(
                lax.bitwise_and(lax.shift_right_logical(uvec, 12), 7), 4)
            og = lax.shift_left(
                lax.bitwise_and(lax.shift_right_logical(ivec, 12), 7), 4)
            pltpu.make_async_copy(up_h.at[pl.ds(0, CH)], bu, sem_g).wait()
            pltpu.make_async_copy(ip_h.at[pl.ds(0, CH)], bi, sem_g).wait()
            pltpu.make_async_copy(uf_h.at[pl.ds(0, CH)], bf, sem_g).wait()
            pltpu.make_async_copy(if_h.at[pl.ds(0, CH)], bg, sem_g).wait()

            @pl.when(jnp.logical_not(first))
            def _():
                pltpu.make_async_copy(
                    slab, out_h.at[pl.ds(pl.multiple_of(base, 8), CH)], sem_s
                ).wait()

            for l in range(CH):
                a = ou[l]
                bq = oi[l]
                f = of[l]
                g = og[l]
                slab[l, pl.ds(0, 16)] = bu[l, pl.ds(a, 16)]
                slab[l, pl.ds(16, 16)] = bu[l, pl.ds(a + 16, 16)]
                slab[l, pl.ds(32, 16)] = bi[l, pl.ds(bq, 16)]
                slab[l, pl.ds(48, 16)] = bi[l, pl.ds(bq + 16, 16)]
                slab[l, pl.ds(64, 16)] = bf[l, pl.ds(f, 16)]
                slab[l, pl.ds(80, 16)] = bg[l, pl.ds(g, 16)]
            row0 = pl.multiple_of(base + c * CH, 8)
            pltpu.async_copy(slab, out_h.at[pl.ds(row0, CH)], sem_s)

        def body(c2, _):
            first = c2 == 0
            fire(0, c2 * 2)
            fire(1, c2 * 2 + 1)
            drain_extract(0, c2 * 2, first)
            drain_extract(1, c2 * 2 + 1, first)
            return 0

        lax.fori_loop(0, NP, body, 0)
        pltpu.make_async_copy(
            slab0, out_h.at[pl.ds(pl.multiple_of(base, 8), CH)], sem_s0
        ).wait()
        pltpu.make_async_copy(
            slab1, out_h.at[pl.ds(pl.multiple_of(base, 8), CH)], sem_s1
        ).wait()

    return k(user_idx, item_idx, up, ip, uf, itf)


def _mlp_body(x, w1, b1, w2, b2, w3, b3, wo3, womf, bo, out):
    xb = x[...]
    h = jnp.dot(xb[:, 0:64], w1[...], preferred_element_type=jnp.float32)
    h = jnp.maximum(h + b1[...], 0.0)
    h = jnp.maximum(jnp.dot(h, w2[...], preferred_element_type=jnp.float32) + b2[...], 0.0)
    h = jnp.maximum(jnp.dot(h, w3[...], preferred_element_type=jnp.float32) + b3[...], 0.0)
    z = jnp.dot(h, wo3[...], preferred_element_type=jnp.float32)
    mf = xb[:, 64:80] * xb[:, 80:96]
    z = z + jnp.dot(mf, womf[...], preferred_element_type=jnp.float32)
    out[...] = jax.nn.sigmoid(z + bo[...])


def _tc_mlp(x, W1, b1, W2, b2, W3, b3, Wo, bo):
    wo3 = Wo[:8]
    womf = Wo[8:]
    b1r = b1.reshape(1, -1)
    b2r = b2.reshape(1, -1)
    b3r = b3.reshape(1, -1)
    bor = bo.reshape(1, -1)

    def full(a):
        return pl.BlockSpec(a.shape, lambda i: (0, 0))

    return pl.pallas_call(
        _mlp_body,
        grid=(B // BLK,),
        in_specs=[
            pl.BlockSpec((BLK, 128), lambda i: (i, 0)),
            full(W1), full(b1r), full(W2), full(b2r),
            full(W3), full(b3r), full(wo3), full(womf), full(bor),
        ],
        out_specs=pl.BlockSpec((BLK, 1), lambda i: (i, 0)),
        out_shape=jax.ShapeDtypeStruct((B, 1), jnp.float32),
    )(x, W1, b1r, W2, b2r, W3, b3r, wo3, womf, bor)


PW = 32768


def _pack_one(xb, out_ref):
    d = xb.shape[0]
    q = 128 // d
    qw = PW // q
    z = jnp.concatenate(
        [xb[:, i * qw:(i + 1) * qw] for i in range(q)], axis=0)
    out_ref[...] = jnp.swapaxes(z, 0, 1)


def _pack_body(xa, xb, outa, outb):
    _pack_one(xa[...], outa)
    _pack_one(xb[...], outb)


def _pack_pair(ta, tb):
    n = ta.shape[0]
    nw = -(-n // PW)
    return pl.pallas_call(
        _pack_body,
        grid=(nw,),
        in_specs=[
            pl.BlockSpec((32, PW), lambda i: (0, i)),
            pl.BlockSpec((16, PW), lambda i: (0, i)),
        ],
        out_specs=[
            pl.BlockSpec((PW * 32 // 128, 128), lambda i: (i, 0)),
            pl.BlockSpec((PW * 16 // 128, 128), lambda i: (i, 0)),
        ],
        out_shape=[
            jax.ShapeDtypeStruct((nw * PW * 32 // 128, 128), jnp.float32),
            jax.ShapeDtypeStruct((nw * PW * 16 // 128, 128), jnp.float32),
        ],
    )(ta.T, tb.T)


def kernel(user_indices, item_indices, U_mlp, I_mlp, U_mf, I_mf,
           W1, b1, W2, b2, W3, b3, Wo, bo):
    up, uf = _pack_pair(U_mlp, U_mf)
    ip, itf = _pack_pair(I_mlp, I_mf)
    x = _sc_gather(user_indices, item_indices, up, ip, uf, itf)
    return _tc_mlp(x, W1, b1, W2, b2, W3, b3, Wo, bo)
